# Initial kernel scaffold; baseline (speedup 1.0000x reference)
import jax
import jax.numpy as jnp
from jax.experimental import pallas as pl


def kernel(input_nodes, input_edges, W_n0, b_n0, W_e0, b_e0, W_n1, b_n1, W_e1, b_e1, edge_index):
    raise NotImplementedError("write your pallas kernel here")


# speedup vs baseline: 2.9329x; 2.9329x over previous
import functools

import jax
import jax.numpy as jnp
from jax import lax
from jax.experimental import pallas as pl
from jax.experimental.pallas import tpu as pltpu
from jax.experimental.pallas import tpu_sc as plsc

N = 10000
E = 320000
DN = 128
DE = 16

_NC = 2
_NS = 16
_NW = _NC * _NS
_EPW = E // _NW
_CH = 80
_NCH = _EPW // _CH

@functools.lru_cache(maxsize=None)
def _sc_gather_fn():
    mesh = plsc.VectorSubcoreMesh(core_axis_name="c", subcore_axis_name="s")

    @functools.partial(
        pl.kernel,
        mesh=mesh,
        out_type=jax.ShapeDtypeStruct((E, DE), jnp.float32),
        scratch_types=[
            pltpu.VMEM((_CH,), jnp.int32),
            pltpu.VMEM((_CH, DE), jnp.float32),
            pltpu.SemaphoreType.DMA,
        ],
        compiler_params=pltpu.CompilerParams(use_tc_tiling_on_sc=False),
    )
    def _sc_gather(p_hbm, src_hbm, out_hbm, idx_v, rows_v, sem):
        c = lax.axis_index("c")
        s = lax.axis_index("s")
        base = (s * _NC + c) * _EPW

        def body(j, carry):
            off = pl.multiple_of(base + j * _CH, 8)
            pltpu.sync_copy(src_hbm.at[pl.ds(off, _CH)], idx_v)
            pltpu.async_copy(p_hbm.at[idx_v], rows_v, sem).wait()
            pltpu.sync_copy(rows_v, out_hbm.at[pl.ds(off, _CH)])
            return carry

        lax.fori_loop(0, _NCH, body, 0)

    return _sc_gather


@functools.lru_cache(maxsize=None)
def _sc_scatter_fn():
    mesh = plsc.VectorSubcoreMesh(core_axis_name="c", subcore_axis_name="s")

    @functools.partial(
        pl.kernel,
        mesh=mesh,
        out_type=jax.ShapeDtypeStruct((_NC, N, DE), jnp.float32),
        scratch_types=[
            pltpu.VMEM((_CH,), jnp.int32),
            pltpu.VMEM((_CH, DE), jnp.float32),
            pltpu.VMEM_SHARED((N, DE), jnp.float32),
        ],
        compiler_params=pltpu.CompilerParams(use_tc_tiling_on_sc=False),
    )
    def _sc_scatter(edges_hbm, dst_hbm, zeros_hbm, out_hbm, idx_v, rows_v, acc_sh):
        c = lax.axis_index("c")
        s = lax.axis_index("s")

        r0 = s * 640

        @pl.when(s < _NS - 1)
        def _():
            pltpu.sync_copy(zeros_hbm.at[pl.ds(r0, 640)], acc_sh.at[pl.ds(r0, 640)])

        @pl.when(s == _NS - 1)
        def _():
            pltpu.sync_copy(zeros_hbm.at[pl.ds(r0, 400)], acc_sh.at[pl.ds(r0, 400)])

        plsc.subcore_barrier()

        base = (c * _NS + s) * _EPW

        def body(j, carry):
            off = pl.multiple_of(base + j * _CH, 8)
            pltpu.sync_copy(dst_hbm.at[pl.ds(off, _CH)], idx_v)
            pltpu.sync_copy(edges_hbm.at[pl.ds(off, _CH)], rows_v)
            pltpu.sync_copy(rows_v, acc_sh.at[idx_v], add=True)
            return carry

        lax.fori_loop(0, _NCH, body, 0)
        plsc.subcore_barrier()

        @pl.when(s < _NS - 1)
        def _():
            pltpu.sync_copy(acc_sh.at[pl.ds(r0, 640)], out_hbm.at[c, pl.ds(r0, 640)])

        @pl.when(s == _NS - 1)
        def _():
            pltpu.sync_copy(acc_sh.at[pl.ds(r0, 400)], out_hbm.at[c, pl.ds(r0, 400)])

    return _sc_scatter


_NBLK = 2000
_EBLK = 2000


def _tc_pre(nodes, wn_top, we_bot):
    def body(x_ref, wt_ref, wb_ref, nw_ref, p_ref):
        x = x_ref[...]
        nw_ref[...] = jnp.dot(x, wt_ref[...], preferred_element_type=jnp.float32)
        p_ref[...] = jnp.dot(x, wb_ref[...], preferred_element_type=jnp.float32)

    return pl.pallas_call(
        body,
        grid=(N // _NBLK,),
        in_specs=[
            pl.BlockSpec((_NBLK, DN), lambda i: (i, 0)),
            pl.BlockSpec((DN, DN), lambda i: (0, 0)),
            pl.BlockSpec((DN, DE), lambda i: (0, 0)),
        ],
        out_specs=[
            pl.BlockSpec((_NBLK, DN), lambda i: (i, 0)),
            pl.BlockSpec((_NBLK, DE), lambda i: (i, 0)),
        ],
        out_shape=[
            jax.ShapeDtypeStruct((N, DN), jnp.float32),
            jax.ShapeDtypeStruct((N, DE), jnp.float32),
        ],
    )(nodes, wn_top, we_bot)


def _tc_node_mid(nw, aggp, wn_bot, bn, wn_top_nxt, we_bot_nxt):
    def body(nw_ref, a_ref, wnb_ref, bn_ref, wt_ref, wb_ref, nw1_ref, p1_ref):
        agg = a_ref[0] + a_ref[1]
        n1 = nw_ref[...] + jnp.dot(agg, wnb_ref[...], preferred_element_type=jnp.float32) + bn_ref[...]
        n1 = jnp.maximum(n1, 0.0)
        nw1_ref[...] = jnp.dot(n1, wt_ref[...], preferred_element_type=jnp.float32)
        p1_ref[...] = jnp.dot(n1, wb_ref[...], preferred_element_type=jnp.float32)

    return pl.pallas_call(
        body,
        grid=(N // _NBLK,),
        in_specs=[
            pl.BlockSpec((_NBLK, DN), lambda i: (i, 0)),
            pl.BlockSpec((_NC, _NBLK, DE), lambda i: (0, i, 0)),
            pl.BlockSpec((DE, DN), lambda i: (0, 0)),
            pl.BlockSpec((1, DN), lambda i: (0, 0)),
            pl.BlockSpec((DN, DN), lambda i: (0, 0)),
            pl.BlockSpec((DN, DE), lambda i: (0, 0)),
        ],
        out_specs=[
            pl.BlockSpec((_NBLK, DN), lambda i: (i, 0)),
            pl.BlockSpec((_NBLK, DE), lambda i: (i, 0)),
        ],
        out_shape=[
            jax.ShapeDtypeStruct((N, DN), jnp.float32),
            jax.ShapeDtypeStruct((N, DE), jnp.float32),
        ],
    )(nw, aggp, wn_bot, bn, wn_top_nxt, we_bot_nxt)


def _tc_node_last(nw, aggp, wn_bot, bn):
    def body(nw_ref, a_ref, wnb_ref, bn_ref, o_ref):
        agg = a_ref[0] + a_ref[1]
        o_ref[...] = nw_ref[...] + jnp.dot(agg, wnb_ref[...], preferred_element_type=jnp.float32) + bn_ref[...]

    return pl.pallas_call(
        body,
        grid=(N // _NBLK,),
        in_specs=[
            pl.BlockSpec((_NBLK, DN), lambda i: (i, 0)),
            pl.BlockSpec((_NC, _NBLK, DE), lambda i: (0, i, 0)),
            pl.BlockSpec((DE, DN), lambda i: (0, 0)),
            pl.BlockSpec((1, DN), lambda i: (0, 0)),
        ],
        out_specs=pl.BlockSpec((_NBLK, DN), lambda i: (i, 0)),
        out_shape=jax.ShapeDtypeStruct((N, DN), jnp.float32),
    )(nw, aggp, wn_bot, bn)


def _tc_edge(edges_r, g_r, bd, be_t, act):
    er = E // 8

    def body(e_ref, g_ref, bd_ref, be_ref, o_ref):
        y = jnp.dot(e_ref[...], bd_ref[...], preferred_element_type=jnp.float32)
        y = y + g_ref[...] + be_ref[...]
        if act:
            y = jnp.maximum(y, 0.0)
        o_ref[...] = y

    return pl.pallas_call(
        body,
        grid=(er // _EBLK,),
        in_specs=[
            pl.BlockSpec((_EBLK, 128), lambda i: (i, 0)),
            pl.BlockSpec((_EBLK, 128), lambda i: (i, 0)),
            pl.BlockSpec((128, 128), lambda i: (0, 0)),
            pl.BlockSpec((1, 128), lambda i: (0, 0)),
        ],
        out_specs=pl.BlockSpec((_EBLK, 128), lambda i: (i, 0)),
        out_shape=jax.ShapeDtypeStruct((er, 128), jnp.float32),
    )(edges_r, g_r, bd, be_t)


def kernel(input_nodes, input_edges, W_n0, b_n0, W_e0, b_e0,
           W_n1, b_n1, W_e1, b_e1, edge_index):
    src = edge_index[0]
    dst = edge_index[1]
    zeros_nd = jnp.zeros((N, DE), jnp.float32)

    wn_top0, wn_bot0 = W_n0[:DN], W_n0[DN:]
    wn_top1, wn_bot1 = W_n1[:DN], W_n1[DN:]
    we_top0, we_bot0 = W_e0[:DE], W_e0[DE:]
    we_top1, we_bot1 = W_e1[:DE], W_e1[DE:]
    eye8 = jnp.eye(8, dtype=jnp.float32)
    bd0 = jnp.kron(eye8, we_top0)
    bd1 = jnp.kron(eye8, we_top1)
    bet0 = jnp.tile(b_e0, 8).reshape(1, 128)
    bet1 = jnp.tile(b_e1, 8).reshape(1, 128)
    bn0 = b_n0.reshape(1, DN)
    bn1 = b_n1.reshape(1, DN)

    edges0_r = input_edges.reshape(E // 8, 128)

    sc_gather = _sc_gather_fn()
    sc_scatter = _sc_scatter_fn()

    nw0, p0 = _tc_pre(input_nodes, wn_top0, we_bot0)
    g0 = sc_gather(p0, src)
    aggp0 = sc_scatter(input_edges, dst, zeros_nd)
    nw1, p1 = _tc_node_mid(nw0, aggp0, wn_bot0, bn0, wn_top1, we_bot1)
    edges1_r = _tc_edge(edges0_r, g0.reshape(E // 8, 128), bd0, bet0, act=True)

    g1 = sc_gather(p1, src)
    aggp1 = sc_scatter(edges1_r.reshape(E, DE), dst, zeros_nd)
    nodes2 = _tc_node_last(nw1, aggp1, wn_bot1, bn1)
    edges2_r = _tc_edge(edges1_r, g1.reshape(E // 8, 128), bd1, bet1, act=False)

    return (nodes2, edges2_r.reshape(E, DE))


# speedup vs baseline: 3.2561x; 1.1102x over previous
import functools

import jax
import jax.numpy as jnp
from jax import lax
from jax.experimental import pallas as pl
from jax.experimental.pallas import tpu as pltpu
from jax.experimental.pallas import tpu_sc as plsc

N = 10000
E = 320000
DN = 128
DE = 16

_NC = 2
_NS = 16
_NW = _NC * _NS
_EPW = E // _NW
_CH = 80
_NCHW = _EPW // _CH
_K = 10
_GR = _K * _CH
_NG = _EPW // _GR


@functools.lru_cache(maxsize=None)
def _sc_layer_fn():
    mesh = plsc.VectorSubcoreMesh(core_axis_name="c", subcore_axis_name="s")

    @functools.partial(
        pl.kernel,
        mesh=mesh,
        out_type=(
            jax.ShapeDtypeStruct((E, DE), jnp.float32),
            jax.ShapeDtypeStruct((_NC, N, DE), jnp.float32),
        ),
        scratch_types=[
            pltpu.VMEM((_NCHW, _CH), jnp.int32),
            pltpu.VMEM((_NCHW, _CH), jnp.int32),
            pltpu.VMEM((_GR, DE), jnp.float32),
            pltpu.VMEM((_GR, DE), jnp.float32),
            pltpu.VMEM((_GR, DE), jnp.float32),
            pltpu.VMEM((_GR, DE), jnp.float32),
            pltpu.VMEM_SHARED((N, DE), jnp.float32),
            pltpu.SemaphoreType.DMA,
            pltpu.SemaphoreType.DMA,
            pltpu.SemaphoreType.DMA,
            pltpu.SemaphoreType.DMA,
            pltpu.SemaphoreType.DMA,
            pltpu.SemaphoreType.DMA,
        ],
        compiler_params=pltpu.CompilerParams(use_tc_tiling_on_sc=False),
    )
    def _sc_layer(p_hbm, ei_hbm, edges_hbm, zeros_hbm, g_hbm, agg_hbm,
                  sidx, didx, gbuf0, gbuf1, ebuf0, ebuf1, acc_sh,
                  gsem, wsem0, wsem1, lsem, asem0, asem1):
        c = lax.axis_index("c")
        s = lax.axis_index("s")
        wid = c * _NS + s
        base = wid * _EPW
        row0 = wid * _NCHW

        gbufs = (gbuf0, gbuf1)
        ebufs = (ebuf0, ebuf1)
        wsems = (wsem0, wsem1)
        asems = (asem0, asem1)

        pltpu.sync_copy(ei_hbm.at[0, pl.ds(row0, _NCHW)], sidx)
        pltpu.sync_copy(ei_hbm.at[1, pl.ds(row0, _NCHW)], didx)

        r0 = s * 640

        @pl.when(s < _NS - 1)
        def _():
            pltpu.sync_copy(zeros_hbm.at[pl.ds(r0, 640)], acc_sh.at[pl.ds(r0, 640)])

        @pl.when(s == _NS - 1)
        def _():
            pltpu.sync_copy(zeros_hbm.at[pl.ds(r0, 400)], acc_sh.at[pl.ds(r0, 400)])

        plsc.subcore_barrier()

        def gather_body(j, carry):
            off = pl.multiple_of(base + j * _CH, 8)
            pltpu.async_copy(p_hbm.at[sidx.at[j]], gbuf0.at[pl.ds(0, _CH)], gsem).wait()
            pltpu.sync_copy(gbuf0.at[pl.ds(0, _CH)], g_hbm.at[pl.ds(off, _CH)])
            return carry

        lax.fori_loop(0, _NCHW, gather_body, 0)

        def scatter_body(j, carry):
            off = pl.multiple_of(base + j * _CH, 8)
            pltpu.sync_copy(edges_hbm.at[pl.ds(off, _CH)], ebuf0.at[pl.ds(0, _CH)])
            pltpu.sync_copy(ebuf0.at[pl.ds(0, _CH)], acc_sh.at[didx.at[j]], add=True)
            return carry

        lax.fori_loop(0, _NCHW, scatter_body, 0)

        plsc.subcore_barrier()

        @pl.when(s < _NS - 1)
        def _():
            pltpu.sync_copy(acc_sh.at[pl.ds(r0, 640)], agg_hbm.at[c, pl.ds(r0, 640)])

        @pl.when(s == _NS - 1)
        def _():
            pltpu.sync_copy(acc_sh.at[pl.ds(r0, 400)], agg_hbm.at[c, pl.ds(r0, 400)])

    return _sc_layer


_NBLK = 2000
_EBLK = 2000


def _tc_pre(nodes, wn_top, we_bot):
    def body(x_ref, wt_ref, wb_ref, nw_ref, p_ref):
        x = x_ref[...]
        nw_ref[...] = jnp.dot(x, wt_ref[...], preferred_element_type=jnp.float32)
        p_ref[...] = jnp.dot(x, wb_ref[...], preferred_element_type=jnp.float32)

    return pl.pallas_call(
        body,
        grid=(N // _NBLK,),
        in_specs=[
            pl.BlockSpec((_NBLK, DN), lambda i: (i, 0)),
            pl.BlockSpec((DN, DN), lambda i: (0, 0)),
            pl.BlockSpec((DN, DE), lambda i: (0, 0)),
        ],
        out_specs=[
            pl.BlockSpec((_NBLK, DN), lambda i: (i, 0)),
            pl.BlockSpec((_NBLK, DE), lambda i: (i, 0)),
        ],
        out_shape=[
            jax.ShapeDtypeStruct((N, DN), jnp.float32),
            jax.ShapeDtypeStruct((N, DE), jnp.float32),
        ],
    )(nodes, wn_top, we_bot)


def _tc_node_mid(nw, aggp, wn_bot, bn, wn_top_nxt, we_bot_nxt):
    def body(nw_ref, a_ref, wnb_ref, bn_ref, wt_ref, wb_ref, nw1_ref, p1_ref):
        agg = a_ref[0] + a_ref[1]
        n1 = nw_ref[...] + jnp.dot(agg, wnb_ref[...], preferred_element_type=jnp.float32) + bn_ref[...]
        n1 = jnp.maximum(n1, 0.0)
        nw1_ref[...] = jnp.dot(n1, wt_ref[...], preferred_element_type=jnp.float32)
        p1_ref[...] = jnp.dot(n1, wb_ref[...], preferred_element_type=jnp.float32)

    return pl.pallas_call(
        body,
        grid=(N // _NBLK,),
        in_specs=[
            pl.BlockSpec((_NBLK, DN), lambda i: (i, 0)),
            pl.BlockSpec((_NC, _NBLK, DE), lambda i: (0, i, 0)),
            pl.BlockSpec((DE, DN), lambda i: (0, 0)),
            pl.BlockSpec((1, DN), lambda i: (0, 0)),
            pl.BlockSpec((DN, DN), lambda i: (0, 0)),
            pl.BlockSpec((DN, DE), lambda i: (0, 0)),
        ],
        out_specs=[
            pl.BlockSpec((_NBLK, DN), lambda i: (i, 0)),
            pl.BlockSpec((_NBLK, DE), lambda i: (i, 0)),
        ],
        out_shape=[
            jax.ShapeDtypeStruct((N, DN), jnp.float32),
            jax.ShapeDtypeStruct((N, DE), jnp.float32),
        ],
    )(nw, aggp, wn_bot, bn, wn_top_nxt, we_bot_nxt)


def _tc_node_last(nw, aggp, wn_bot, bn):
    def body(nw_ref, a_ref, wnb_ref, bn_ref, o_ref):
        agg = a_ref[0] + a_ref[1]
        o_ref[...] = nw_ref[...] + jnp.dot(agg, wnb_ref[...], preferred_element_type=jnp.float32) + bn_ref[...]

    return pl.pallas_call(
        body,
        grid=(N // _NBLK,),
        in_specs=[
            pl.BlockSpec((_NBLK, DN), lambda i: (i, 0)),
            pl.BlockSpec((_NC, _NBLK, DE), lambda i: (0, i, 0)),
            pl.BlockSpec((DE, DN), lambda i: (0, 0)),
            pl.BlockSpec((1, DN), lambda i: (0, 0)),
        ],
        out_specs=pl.BlockSpec((_NBLK, DN), lambda i: (i, 0)),
        out_shape=jax.ShapeDtypeStruct((N, DN), jnp.float32),
    )(nw, aggp, wn_bot, bn)


def _tc_edge(edges_r, g_r, bd, be_t, act):
    er = E // 8

    def body(e_ref, g_ref, bd_ref, be_ref, o_ref):
        y = jnp.dot(e_ref[...], bd_ref[...], preferred_element_type=jnp.float32)
        y = y + g_ref[...] + be_ref[...]
        if act:
            y = jnp.maximum(y, 0.0)
        o_ref[...] = y

    return pl.pallas_call(
        body,
        grid=(er // _EBLK,),
        in_specs=[
            pl.BlockSpec((_EBLK, 128), lambda i: (i, 0)),
            pl.BlockSpec((_EBLK, 128), lambda i: (i, 0)),
            pl.BlockSpec((128, 128), lambda i: (0, 0)),
            pl.BlockSpec((1, 128), lambda i: (0, 0)),
        ],
        out_specs=pl.BlockSpec((_EBLK, 128), lambda i: (i, 0)),
        out_shape=jax.ShapeDtypeStruct((er, 128), jnp.float32),
    )(edges_r, g_r, bd, be_t)


def kernel(input_nodes, input_edges, W_n0, b_n0, W_e0, b_e0,
           W_n1, b_n1, W_e1, b_e1, edge_index):
    ei_r = edge_index.reshape(2, E // _CH, _CH)
    zeros_nd = jnp.zeros((N, DE), jnp.float32)

    wn_top0, wn_bot0 = W_n0[:DN], W_n0[DN:]
    wn_top1, wn_bot1 = W_n1[:DN], W_n1[DN:]
    we_top0, we_bot0 = W_e0[:DE], W_e0[DE:]
    we_top1, we_bot1 = W_e1[:DE], W_e1[DE:]
    eye8 = jnp.eye(8, dtype=jnp.float32)
    bd0 = jnp.kron(eye8, we_top0)
    bd1 = jnp.kron(eye8, we_top1)
    bet0 = jnp.tile(b_e0, 8).reshape(1, 128)
    bet1 = jnp.tile(b_e1, 8).reshape(1, 128)
    bn0 = b_n0.reshape(1, DN)
    bn1 = b_n1.reshape(1, DN)

    edges0_r = input_edges.reshape(E // 8, 128)
    sc_layer = _sc_layer_fn()

    nw0, p0 = _tc_pre(input_nodes, wn_top0, we_bot0)
    g0, aggp0 = sc_layer(p0, ei_r, input_edges, zeros_nd)
    nw1, p1 = _tc_node_mid(nw0, aggp0, wn_bot0, bn0, wn_top1, we_bot1)
    edges1_r = _tc_edge(edges0_r, g0.reshape(E // 8, 128), bd0, bet0, act=True)

    g1, aggp1 = sc_layer(p1, ei_r, edges1_r.reshape(E, DE), zeros_nd)
    nodes2 = _tc_node_last(nw1, aggp1, wn_bot1, bn1)
    edges2_r = _tc_edge(edges1_r, g1.reshape(E // 8, 128), bd1, bet1, act=False)

    return (nodes2, edges2_r.reshape(E, DE))


# speedup vs baseline: 4.7470x; 1.4579x over previous
import functools

import jax
import jax.numpy as jnp
from jax import lax
from jax.experimental import pallas as pl
from jax.experimental.pallas import tpu as pltpu
from jax.experimental.pallas import tpu_sc as plsc

N = 10000
E = 320000
DN = 128
DE = 16

_NC = 2
_NS = 16
_NW = _NC * _NS
_EPW = E // _NW
_CH = 80
_M = _EPW // _CH
_R = 5
_CR = _CH // 8


@functools.lru_cache(maxsize=None)
def _sc_layer_fn():
    mesh = plsc.VectorSubcoreMesh(core_axis_name="c", subcore_axis_name="s")

    @functools.partial(
        pl.kernel,
        mesh=mesh,
        out_type=(
            jax.ShapeDtypeStruct((E // 8, 128), jnp.float32),
            jax.ShapeDtypeStruct((_NC, N // 8, 128), jnp.float32),
        ),
        scratch_types=(
            [
                pltpu.VMEM((_M, _CH), jnp.int32),
                pltpu.VMEM((_M, _CH), jnp.int32),
                pltpu.VMEM_SHARED((N, DE), jnp.float32),
                pltpu.VMEM((640, DE), jnp.float32),
                pltpu.VMEM((80, 128), jnp.float32),
            ]
            + [pltpu.VMEM((_CH, DE), jnp.float32)] * _R
            + [pltpu.VMEM((_CR, 128), jnp.float32)] * _R
            + [pltpu.VMEM((_CR, 128), jnp.float32)] * _R
            + [pltpu.VMEM((_CH, DE), jnp.float32)] * _R
            + [pltpu.SemaphoreType.DMA] * (4 * _R)
        ),
        compiler_params=pltpu.CompilerParams(use_tc_tiling_on_sc=False),
    )
    def _sc_layer(p_hbm, ei_hbm, edges_hbm, g_hbm, agg_hbm,
                  sidx, didx, acc_sh, vb, vp, *bufs_sems):
        c = lax.axis_index("c")
        s = lax.axis_index("s")
        wid = c * _NS + s
        row0 = wid * _M

        gbufs = bufs_sems[0:_R]
        gpacks = bufs_sems[_R:2 * _R]
        epacks = bufs_sems[2 * _R:3 * _R]
        ebufs = bufs_sems[3 * _R:4 * _R]
        gsems = bufs_sems[4 * _R:5 * _R]
        wsems = bufs_sems[5 * _R:6 * _R]
        lsems = bufs_sems[6 * _R:7 * _R]
        asems = bufs_sems[7 * _R:8 * _R]

        pltpu.sync_copy(ei_hbm.at[0, pl.ds(row0, _M)], sidx)
        pltpu.sync_copy(ei_hbm.at[1, pl.ds(row0, _M)], didx)

        zero16 = jnp.zeros((16,), jnp.float32)

        def zrow(i, carry):
            vb[i, :] = zero16
            return carry

        lax.fori_loop(0, 640, zrow, 0)
        r0 = s * 640

        @pl.when(s < _NS - 1)
        def _():
            pltpu.sync_copy(vb, acc_sh.at[pl.ds(r0, 640)])

        @pl.when(s == _NS - 1)
        def _():
            pltpu.sync_copy(vb.at[pl.ds(0, 400)], acc_sh.at[pl.ds(r0, 400)])

        plsc.subcore_barrier()

        def fire_gather(j, r):
            pltpu.async_copy(p_hbm.at[sidx.at[j]], gbufs[r], gsems[r])

        def drain_gather(j, r):
            pltpu.make_async_copy(p_hbm.at[sidx.at[j]], gbufs[r], gsems[r]).wait()

        def repack_g(r):
            def qbody(q, carry):
                for i in range(8):
                    gpacks[r][q, pl.ds(i * DE, DE)] = gbufs[r][8 * q + i, :]
                return carry

            lax.fori_loop(0, _CR, qbody, 0)

        def fire_wb(j, r):
            pltpu.async_copy(gpacks[r], g_hbm.at[pl.ds((row0 + j) * _CR, _CR)],
                             wsems[r])

        def wait_wb(j, r):
            pltpu.make_async_copy(gpacks[r],
                                  g_hbm.at[pl.ds((row0 + j) * _CR, _CR)],
                                  wsems[r]).wait()

        for r in range(_R):
            fire_gather(r, r)

        def gather_body(m, carry):
            for r in range(_R):
                j = m * _R + r
                drain_gather(j, r)

                @pl.when(m > 0)
                def _():
                    wait_wb(j - _R, r)

                repack_g(r)
                fire_wb(j, r)

                @pl.when(m < _M // _R - 1)
                def _():
                    fire_gather(j + _R, r)
            return carry

        lax.fori_loop(0, _M // _R, gather_body, 0)
        for r in range(_R):
            wait_wb(_M - _R + r, r)

        def fire_load(j, r):
            pltpu.async_copy(edges_hbm.at[pl.ds((row0 + j) * _CR, _CR)],
                             epacks[r], lsems[r])

        def wait_load(j, r):
            pltpu.make_async_copy(edges_hbm.at[pl.ds((row0 + j) * _CR, _CR)],
                                  epacks[r], lsems[r]).wait()

        def repack_e(r):
            def qbody(q, carry):
                for i in range(8):
                    ebufs[r][8 * q + i, :] = epacks[r][q, pl.ds(i * DE, DE)]
                return carry

            lax.fori_loop(0, _CR, qbody, 0)

        def fire_add(j, r):
            pltpu.async_copy(ebufs[r], acc_sh.at[didx.at[j]], asems[r], add=True)

        def drain_add(j, r):
            pltpu.make_async_copy(ebufs[r], acc_sh.at[didx.at[j]], asems[r]).wait()

        for r in range(_R):
            fire_load(r, r)

        def scatter_body(m, carry):
            for r in range(_R):
                j = m * _R + r
                wait_load(j, r)

                @pl.when(m > 0)
                def _():
                    drain_add(j - _R, r)

                repack_e(r)
                fire_add(j, r)

                @pl.when(m < _M // _R - 1)
                def _():
                    fire_load(j + _R, r)
            return carry

        lax.fori_loop(0, _M // _R, scatter_body, 0)
        for r in range(_R):
            drain_add(_M - _R + r, r)

        plsc.subcore_barrier()

        @pl.when(s < _NS - 1)
        def _():
            pltpu.sync_copy(acc_sh.at[pl.ds(r0, 640)], vb)

        @pl.when(s == _NS - 1)
        def _():
            pltpu.sync_copy(acc_sh.at[pl.ds(r0, 400)], vb.at[pl.ds(0, 400)])

        def pub_row(q, carry):
            for i in range(8):
                vp[q, pl.ds(i * DE, DE)] = vb[8 * q + i, :]
            return carry

        nrow = jnp.where(s == _NS - 1, 50, 80)
        lax.fori_loop(0, nrow, pub_row, 0)

        @pl.when(s < _NS - 1)
        def _():
            pltpu.sync_copy(vp, agg_hbm.at[c, pl.ds(s * 80, 80)])

        @pl.when(s == _NS - 1)
        def _():
            pltpu.sync_copy(vp.at[pl.ds(0, 50)], agg_hbm.at[c, pl.ds(s * 80, 50)])

    return _sc_layer


_NBLK = 2000
_EBLK = 2000
_ABLK = N // 8


def _tc_pre(nodes, wn_top, we_bot):
    def body(x_ref, wt_ref, wb_ref, nw_ref, p_ref):
        x = x_ref[...]
        nw_ref[...] = jnp.dot(x, wt_ref[...], preferred_element_type=jnp.float32)
        p_ref[...] = jnp.dot(x, wb_ref[...], preferred_element_type=jnp.float32)

    return pl.pallas_call(
        body,
        grid=(N // _NBLK,),
        in_specs=[
            pl.BlockSpec((_NBLK, DN), lambda i: (i, 0)),
            pl.BlockSpec((DN, DN), lambda i: (0, 0)),
            pl.BlockSpec((DN, DE), lambda i: (0, 0)),
        ],
        out_specs=[
            pl.BlockSpec((_NBLK, DN), lambda i: (i, 0)),
            pl.BlockSpec((_NBLK, DE), lambda i: (i, 0)),
        ],
        out_shape=[
            jax.ShapeDtypeStruct((N, DN), jnp.float32),
            jax.ShapeDtypeStruct((N, DE), jnp.float32),
        ],
    )(nodes, wn_top, we_bot)


def _node_update(nw_ref, a_ref, kn_ref, bn_ref, act):
    sagg = a_ref[0] + a_ref[1]
    y = jnp.dot(sagg, kn_ref[...], preferred_element_type=jnp.float32)
    y = y.reshape(N, DN)
    out = nw_ref[...] + y + bn_ref[...]
    if act:
        out = jnp.maximum(out, 0.0)
    return out


def _tc_node_mid(nw, aggp, kn, bn, wn_top_nxt, we_bot_nxt):
    def body(nw_ref, a_ref, kn_ref, bn_ref, wt_ref, wb_ref, nw1_ref, p1_ref):
        n1 = _node_update(nw_ref, a_ref, kn_ref, bn_ref, act=True)
        nw1_ref[...] = jnp.dot(n1, wt_ref[...], preferred_element_type=jnp.float32)
        p1_ref[...] = jnp.dot(n1, wb_ref[...], preferred_element_type=jnp.float32)

    return pl.pallas_call(
        body,
        out_shape=[
            jax.ShapeDtypeStruct((N, DN), jnp.float32),
            jax.ShapeDtypeStruct((N, DE), jnp.float32),
        ],
    )(nw, aggp, kn, bn, wn_top_nxt, we_bot_nxt)


def _tc_node_last(nw, aggp, kn, bn):
    def body(nw_ref, a_ref, kn_ref, bn_ref, o_ref):
        o_ref[...] = _node_update(nw_ref, a_ref, kn_ref, bn_ref, act=False)

    return pl.pallas_call(
        body,
        out_shape=jax.ShapeDtypeStruct((N, DN), jnp.float32),
    )(nw, aggp, kn, bn)


def _tc_edge(edges_r, g_r, bd, be_t, act):
    er = E // 8

    def body(e_ref, g_ref, bd_ref, be_ref, o_ref):
        y = jnp.dot(e_ref[...], bd_ref[...], preferred_element_type=jnp.float32)
        y = y + g_ref[...] + be_ref[...]
        if act:
            y = jnp.maximum(y, 0.0)
        o_ref[...] = y

    return pl.pallas_call(
        body,
        grid=(er // _EBLK,),
        in_specs=[
            pl.BlockSpec((_EBLK, 128), lambda i: (i, 0)),
            pl.BlockSpec((_EBLK, 128), lambda i: (i, 0)),
            pl.BlockSpec((128, 128), lambda i: (0, 0)),
            pl.BlockSpec((1, 128), lambda i: (0, 0)),
        ],
        out_specs=pl.BlockSpec((_EBLK, 128), lambda i: (i, 0)),
        out_shape=jax.ShapeDtypeStruct((er, 128), jnp.float32),
    )(edges_r, g_r, bd, be_t)


def kernel(input_nodes, input_edges, W_n0, b_n0, W_e0, b_e0,
           W_n1, b_n1, W_e1, b_e1, edge_index):
    ei_r = edge_index.reshape(2, E // _CH, _CH)

    wn_top0, wn_bot0 = W_n0[:DN], W_n0[DN:]
    wn_top1, wn_bot1 = W_n1[:DN], W_n1[DN:]
    we_top0, we_bot0 = W_e0[:DE], W_e0[DE:]
    we_top1, we_bot1 = W_e1[:DE], W_e1[DE:]
    eye8 = jnp.eye(8, dtype=jnp.float32)
    bd0 = jnp.kron(eye8, we_top0)
    bd1 = jnp.kron(eye8, we_top1)
    kn0 = jnp.kron(eye8, wn_bot0)
    kn1 = jnp.kron(eye8, wn_bot1)
    bet0 = jnp.tile(b_e0, 8).reshape(1, 128)
    bet1 = jnp.tile(b_e1, 8).reshape(1, 128)
    bn0 = b_n0.reshape(1, DN)
    bn1 = b_n1.reshape(1, DN)

    edges0_r = input_edges.reshape(E // 8, 128)
    sc_layer = _sc_layer_fn()

    nw0, p0 = _tc_pre(input_nodes, wn_top0, we_bot0)
    g0, aggp0 = sc_layer(p0, ei_r, edges0_r)
    nw1, p1 = _tc_node_mid(nw0, aggp0, kn0, bn0, wn_top1, we_bot1)
    edges1_r = _tc_edge(edges0_r, g0, bd0, bet0, act=True)

    g1, aggp1 = sc_layer(p1, ei_r, edges1_r)
    nodes2 = _tc_node_last(nw1, aggp1, kn1, bn1)
    edges2_r = _tc_edge(edges1_r, g1, bd1, bet1, act=False)

    return (nodes2, edges2_r.reshape(E, DE))


# speedup vs baseline: 4.9759x; 1.0482x over previous
import functools

import jax
import jax.numpy as jnp
from jax import lax
from jax.experimental import pallas as pl
from jax.experimental.pallas import tpu as pltpu
from jax.experimental.pallas import tpu_sc as plsc

N = 10000
E = 320000
DN = 128
DE = 16

_NC = 2
_NS = 16
_NW = _NC * _NS
_EPW = E // _NW
_CH = 80
_M = _EPW // _CH
_R = 5
_CR = _CH // 8


@functools.lru_cache(maxsize=None)
def _sc_gather_fn():
    mesh = plsc.VectorSubcoreMesh(core_axis_name="c", subcore_axis_name="s")

    @functools.partial(
        pl.kernel,
        mesh=mesh,
        out_type=jax.ShapeDtypeStruct((E // 8, 128), jnp.float32),
        scratch_types=(
            [pltpu.VMEM((_M, _CH), jnp.int32)]
            + [pltpu.VMEM((_CH, DE), jnp.float32)] * _R
            + [pltpu.VMEM((_CR, 128), jnp.float32)] * _R
            + [pltpu.SemaphoreType.DMA] * (2 * _R)
        ),
        compiler_params=pltpu.CompilerParams(use_tc_tiling_on_sc=False),
    )
    def _sc_gather(p_hbm, ei_hbm, g_hbm, sidx, *bufs_sems):
        c = lax.axis_index("c")
        s = lax.axis_index("s")
        wid = c * _NS + s
        row0 = wid * _M

        gbufs = bufs_sems[0:_R]
        gpacks = bufs_sems[_R:2 * _R]
        gsems = bufs_sems[2 * _R:3 * _R]
        wsems = bufs_sems[3 * _R:4 * _R]

        pltpu.sync_copy(ei_hbm.at[0, pl.ds(row0, _M)], sidx)

        def fire_gather(j, r):
            pltpu.async_copy(p_hbm.at[sidx.at[j]], gbufs[r], gsems[r])

        def drain_gather(j, r):
            pltpu.make_async_copy(p_hbm.at[sidx.at[j]], gbufs[r], gsems[r]).wait()

        def repack_g(r):
            def qbody(q, carry):
                for i in range(8):
                    gpacks[r][q, pl.ds(i * DE, DE)] = gbufs[r][8 * q + i, :]
                return carry

            lax.fori_loop(0, _CR, qbody, 0)

        def fire_wb(j, r):
            pltpu.async_copy(gpacks[r], g_hbm.at[pl.ds((row0 + j) * _CR, _CR)],
                             wsems[r])

        def wait_wb(j, r):
            pltpu.make_async_copy(gpacks[r],
                                  g_hbm.at[pl.ds((row0 + j) * _CR, _CR)],
                                  wsems[r]).wait()

        for r in range(_R):
            fire_gather(r, r)

        def gather_body(m, carry):
            for r in range(_R):
                j = m * _R + r
                drain_gather(j, r)

                @pl.when(m > 0)
                def _():
                    wait_wb(j - _R, r)

                repack_g(r)
                fire_wb(j, r)

                @pl.when(m < _M // _R - 1)
                def _():
                    fire_gather(j + _R, r)
            return carry

        lax.fori_loop(0, _M // _R, gather_body, 0)
        for r in range(_R):
            wait_wb(_M - _R + r, r)

    return _sc_gather


@functools.lru_cache(maxsize=None)
def _sc_scatter_fn():
    mesh = plsc.VectorSubcoreMesh(core_axis_name="c", subcore_axis_name="s")

    @functools.partial(
        pl.kernel,
        mesh=mesh,
        out_type=jax.ShapeDtypeStruct((_NC, N // 8, 128), jnp.float32),
        scratch_types=(
            [
                pltpu.VMEM((_M, _CH), jnp.int32),
                pltpu.VMEM_SHARED((N, DE), jnp.float32),
                pltpu.VMEM((640, DE), jnp.float32),
                pltpu.VMEM((80, 128), jnp.float32),
            ]
            + [pltpu.VMEM((_CR, 128), jnp.float32)] * _R
            + [pltpu.VMEM((_CH, DE), jnp.float32)] * _R
            + [pltpu.SemaphoreType.DMA] * (2 * _R)
        ),
        compiler_params=pltpu.CompilerParams(use_tc_tiling_on_sc=False),
    )
    def _sc_scatter(ei_hbm, edges_hbm, agg_hbm, didx, acc_sh, vb, vp,
                    *bufs_sems):
        c = lax.axis_index("c")
        s = lax.axis_index("s")
        wid = c * _NS + s
        row0 = wid * _M

        epacks = bufs_sems[0:_R]
        ebufs = bufs_sems[_R:2 * _R]
        lsems = bufs_sems[2 * _R:3 * _R]
        asems = bufs_sems[3 * _R:4 * _R]

        pltpu.sync_copy(ei_hbm.at[1, pl.ds(row0, _M)], didx)

        zero16 = jnp.zeros((16,), jnp.float32)

        def zrow(i, carry):
            vb[i, :] = zero16
            return carry

        lax.fori_loop(0, 640, zrow, 0)
        r0 = s * 640

        @pl.when(s < _NS - 1)
        def _():
            pltpu.sync_copy(vb, acc_sh.at[pl.ds(r0, 640)])

        @pl.when(s == _NS - 1)
        def _():
            pltpu.sync_copy(vb.at[pl.ds(0, 400)], acc_sh.at[pl.ds(r0, 400)])

        plsc.subcore_barrier()

        def fire_load(j, r):
            pltpu.async_copy(edges_hbm.at[pl.ds((row0 + j) * _CR, _CR)],
                             epacks[r], lsems[r])

        def wait_load(j, r):
            pltpu.make_async_copy(edges_hbm.at[pl.ds((row0 + j) * _CR, _CR)],
                                  epacks[r], lsems[r]).wait()

        def repack_e(r):
            def qbody(q, carry):
                for i in range(8):
                    ebufs[r][8 * q + i, :] = epacks[r][q, pl.ds(i * DE, DE)]
                return carry

            lax.fori_loop(0, _CR, qbody, 0)

        def fire_add(j, r):
            pltpu.async_copy(ebufs[r], acc_sh.at[didx.at[j]], asems[r], add=True)

        def drain_add(j, r):
            pltpu.make_async_copy(ebufs[r], acc_sh.at[didx.at[j]], asems[r]).wait()

        for r in range(_R):
            fire_load(r, r)

        def scatter_body(m, carry):
            for r in range(_R):
                j = m * _R + r
                wait_load(j, r)

                @pl.when(m > 0)
                def _():
                    drain_add(j - _R, r)

                repack_e(r)
                fire_add(j, r)

                @pl.when(m < _M // _R - 1)
                def _():
                    fire_load(j + _R, r)
            return carry

        lax.fori_loop(0, _M // _R, scatter_body, 0)
        for r in range(_R):
            drain_add(_M - _R + r, r)

        plsc.subcore_barrier()

        @pl.when(s < _NS - 1)
        def _():
            pltpu.sync_copy(acc_sh.at[pl.ds(r0, 640)], vb)

        @pl.when(s == _NS - 1)
        def _():
            pltpu.sync_copy(acc_sh.at[pl.ds(r0, 400)], vb.at[pl.ds(0, 400)])

        def pub_row(q, carry):
            for i in range(8):
                vp[q, pl.ds(i * DE, DE)] = vb[8 * q + i, :]
            return carry

        nrow = jnp.where(s == _NS - 1, 50, 80)
        lax.fori_loop(0, nrow, pub_row, 0)

        @pl.when(s < _NS - 1)
        def _():
            pltpu.sync_copy(vp, agg_hbm.at[c, pl.ds(s * 80, 80)])

        @pl.when(s == _NS - 1)
        def _():
            pltpu.sync_copy(vp.at[pl.ds(0, 50)], agg_hbm.at[c, pl.ds(s * 80, 50)])

    return _sc_scatter


_NBLK = 2000
_EBLK = 2000
_ABLK = N // 8


def _tc_pre(nodes, wn_top, we_bot):
    def body(x_ref, wt_ref, wb_ref, nw_ref, p_ref):
        x = x_ref[...]
        nw_ref[...] = jnp.dot(x, wt_ref[...], preferred_element_type=jnp.float32)
        p_ref[...] = jnp.dot(x, wb_ref[...], preferred_element_type=jnp.float32)

    return pl.pallas_call(
        body,
        grid=(N // _NBLK,),
        in_specs=[
            pl.BlockSpec((_NBLK, DN), lambda i: (i, 0)),
            pl.BlockSpec((DN, DN), lambda i: (0, 0)),
            pl.BlockSpec((DN, DE), lambda i: (0, 0)),
        ],
        out_specs=[
            pl.BlockSpec((_NBLK, DN), lambda i: (i, 0)),
            pl.BlockSpec((_NBLK, DE), lambda i: (i, 0)),
        ],
        out_shape=[
            jax.ShapeDtypeStruct((N, DN), jnp.float32),
            jax.ShapeDtypeStruct((N, DE), jnp.float32),
        ],
    )(nodes, wn_top, we_bot)


def _node_update(nw_ref, a_ref, kn_ref, bn_ref, act):
    sagg = a_ref[0] + a_ref[1]
    y = jnp.dot(sagg, kn_ref[...], preferred_element_type=jnp.float32)
    y = y.reshape(N, DN)
    out = nw_ref[...] + y + bn_ref[...]
    if act:
        out = jnp.maximum(out, 0.0)
    return out


def _tc_node_mid(nw, aggp, kn, bn, wn_top_nxt, we_bot_nxt):
    def body(nw_ref, a_ref, kn_ref, bn_ref, wt_ref, wb_ref, nw1_ref, p1_ref):
        n1 = _node_update(nw_ref, a_ref, kn_ref, bn_ref, act=True)
        nw1_ref[...] = jnp.dot(n1, wt_ref[...], preferred_element_type=jnp.float32)
        p1_ref[...] = jnp.dot(n1, wb_ref[...], preferred_element_type=jnp.float32)

    return pl.pallas_call(
        body,
        out_shape=[
            jax.ShapeDtypeStruct((N, DN), jnp.float32),
            jax.ShapeDtypeStruct((N, DE), jnp.float32),
        ],
    )(nw, aggp, kn, bn, wn_top_nxt, we_bot_nxt)


def _tc_node_last(nw, aggp, kn, bn):
    def body(nw_ref, a_ref, kn_ref, bn_ref, o_ref):
        o_ref[...] = _node_update(nw_ref, a_ref, kn_ref, bn_ref, act=False)

    return pl.pallas_call(
        body,
        out_shape=jax.ShapeDtypeStruct((N, DN), jnp.float32),
    )(nw, aggp, kn, bn)


def _tc_edge(edges_r, g_r, bd, be_t, act):
    er = E // 8

    def body(e_ref, g_ref, bd_ref, be_ref, o_ref):
        y = jnp.dot(e_ref[...], bd_ref[...], preferred_element_type=jnp.float32)
        y = y + g_ref[...] + be_ref[...]
        if act:
            y = jnp.maximum(y, 0.0)
        o_ref[...] = y

    return pl.pallas_call(
        body,
        grid=(er // _EBLK,),
        in_specs=[
            pl.BlockSpec((_EBLK, 128), lambda i: (i, 0)),
            pl.BlockSpec((_EBLK, 128), lambda i: (i, 0)),
            pl.BlockSpec((128, 128), lambda i: (0, 0)),
            pl.BlockSpec((1, 128), lambda i: (0, 0)),
        ],
        out_specs=pl.BlockSpec((_EBLK, 128), lambda i: (i, 0)),
        out_shape=jax.ShapeDtypeStruct((er, 128), jnp.float32),
    )(edges_r, g_r, bd, be_t)


def kernel(input_nodes, input_edges, W_n0, b_n0, W_e0, b_e0,
           W_n1, b_n1, W_e1, b_e1, edge_index):
    ei_r = edge_index.reshape(2, E // _CH, _CH)

    wn_top0, wn_bot0 = W_n0[:DN], W_n0[DN:]
    wn_top1, wn_bot1 = W_n1[:DN], W_n1[DN:]
    we_top0, we_bot0 = W_e0[:DE], W_e0[DE:]
    we_top1, we_bot1 = W_e1[:DE], W_e1[DE:]
    eye8 = jnp.eye(8, dtype=jnp.float32)
    bd0 = jnp.kron(eye8, we_top0)
    bd1 = jnp.kron(eye8, we_top1)
    kn0 = jnp.kron(eye8, wn_bot0)
    kn1 = jnp.kron(eye8, wn_bot1)
    bet0 = jnp.tile(b_e0, 8).reshape(1, 128)
    bet1 = jnp.tile(b_e1, 8).reshape(1, 128)
    bn0 = b_n0.reshape(1, DN)
    bn1 = b_n1.reshape(1, DN)

    edges0_r = input_edges.reshape(E // 8, 128)
    sc_gather = _sc_gather_fn()
    sc_scatter = _sc_scatter_fn()

    nw0, p0 = _tc_pre(input_nodes, wn_top0, we_bot0)
    g0 = sc_gather(p0, ei_r)
    aggp0 = sc_scatter(ei_r, edges0_r)
    nw1, p1 = _tc_node_mid(nw0, aggp0, kn0, bn0, wn_top1, we_bot1)
    edges1_r = _tc_edge(edges0_r, g0, bd0, bet0, act=True)

    g1 = sc_gather(p1, ei_r)
    aggp1 = sc_scatter(ei_r, edges1_r)
    nodes2 = _tc_node_last(nw1, aggp1, kn1, bn1)
    edges2_r = _tc_edge(edges1_r, g1, bd1, bet1, act=False)

    return (nodes2, edges2_r.reshape(E, DE))
